# Initial kernel scaffold; baseline (speedup 1.0000x reference)
import jax
import jax.numpy as jnp
from jax.experimental import pallas as pl


def kernel(coords, feats, params):
    raise NotImplementedError("write your pallas kernel here")


# speedup vs baseline: 4.1944x; 4.1944x over previous
import functools

import numpy as np
import jax
import jax.numpy as jnp
from jax import lax
from jax.experimental import pallas as pl
from jax.experimental.pallas import tpu as pltpu

F32 = jnp.float32
_BN = float(1.0 / np.sqrt(1.0 + 1e-4))
_HI = lax.Precision.HIGHEST

_SA_CFG = ((1024, 0.1, 32), (256, 0.2, 32), (64, 0.4, 32), (16, 0.8, 32))
_SA1_BLOCK = 64
_FP1_BLOCK = 1024


def _mm(a, b, prec=None):
    return lax.dot_general(a, b, (((1,), (0,)), ((), ())), precision=prec,
                           preferred_element_type=F32)


def _mm_t(a, b, prec=None):
    return lax.dot_general(a, b, (((1,), (1,)), ((), ())), precision=prec,
                           preferred_element_type=F32)


def _mlp(x, wbs, final_act=True):
    n = len(wbs)
    for i, (w, b) in enumerate(wbs):
        x = _mm_t(x, w) + b
        if final_act or i < n - 1:
            x = jnp.maximum(x * _BN, 0.0)
    return x


def _fps_body(xyz, npoint):
    N = xyz.shape[1]
    iota_n = lax.broadcasted_iota(jnp.int32, (1, N), 1)
    iota_p = lax.broadcasted_iota(jnp.int32, (1, npoint), 1)

    def body(i, state):
        dist, far, acc = state
        oh = jnp.where(iota_n == far, 1.0, 0.0).astype(F32)
        c = _mm_t(xyz, oh, _HI)
        acc = jnp.where(iota_p == i, c, acc)
        d = jnp.sum((xyz - c) ** 2, axis=0, keepdims=True)
        dist = jnp.minimum(dist, d)
        m = jnp.max(dist, axis=1, keepdims=True)
        far = jnp.min(jnp.where(dist == m, iota_n, N), axis=1, keepdims=True)
        return dist, far, acc

    state = (jnp.full((1, N), 1e10, F32), jnp.zeros((1, 1), jnp.int32),
             jnp.zeros((3, npoint), F32))
    _, _, acc = lax.fori_loop(0, npoint, body, state)
    return acc


def _ball_gather(d2, radius, nsample, slot_fn):
    S, N = d2.shape
    iota = lax.broadcasted_iota(jnp.int32, (S, N), 1)
    midx0 = jnp.where(d2 <= radius * radius, iota, N)

    def slot(k, midx):
        m = jnp.min(midx, axis=1, keepdims=True)
        sel = midx == m
        midx = jnp.where(sel, N, midx)
        slot_fn(k, m < N, jnp.where(sel, 1.0, 0.0).astype(F32))
        return midx

    lax.fori_loop(0, nsample, slot, midx0)


def _sqdists(aT, bT):
    a2 = jnp.sum(aT * aT, axis=1, keepdims=True)
    b2 = jnp.sum(bT * bT, axis=0, keepdims=True)
    return a2 + b2 - 2.0 * _mm(aT, bT)


def _interp3(x1t, x2t, P2):
    x1 = jnp.transpose(x1t)
    d2 = _sqdists(x1, x2t)
    S, N2 = d2.shape
    iota = lax.broadcasted_iota(jnp.int32, (S, N2), 1)
    fs, ws = [], []
    for _ in range(3):
        m = jnp.min(d2, axis=1, keepdims=True)
        first = jnp.min(jnp.where(d2 == m, iota, N2), axis=1, keepdims=True)
        oh = iota == first
        d2 = jnp.where(oh, 3.0e38, d2)
        ws.append(1.0 / (jnp.maximum(m, 0.0) + 1e-8))
        fs.append(_mm(jnp.where(oh, 1.0, 0.0).astype(F32), P2, _HI))
    wsum = (ws[0] + ws[1]) + ws[2]
    return (fs[0] * (ws[0] / wsum) + fs[1] * (ws[1] / wsum)) + fs[2] * (ws[2] / wsum)


def _fps1_kernel(xt_ref, out_ref, *, npoint):
    out_ref[0] = _fps_body(xt_ref[0], npoint)


def _sa1_kernel(xt_ref, nxt_ref, *refs, radius, nsample, nlayers):
    prs = refs[:2 * nlayers]
    out_ref, g_ref = refs[2 * nlayers], refs[2 * nlayers + 1]
    xyz = xt_ref[0]
    nxT = nxt_ref[0]
    S = nxT.shape[0]
    d2 = _sqdists(nxT, xyz)

    def slot(k, has, sel_f):
        gT = jnp.transpose(_mm_t(xyz, sel_f, _HI)) - nxT
        g_ref[k] = jnp.where(has, gT, g_ref[0])

    _ball_gather(d2, radius, nsample, slot)
    X = g_ref[...].reshape(nsample * S, 3)
    wbs = [(prs[2 * i][...], prs[2 * i + 1][...]) for i in range(nlayers)]
    h = _mlp(X, wbs, True)
    out_ref[0] = jnp.max(h.reshape(nsample, S, h.shape[1]), axis=0)


def _sa_full_kernel(xt_ref, f_ref, *refs, npoint, radius, nsample, nlayers):
    prs = refs[:2 * nlayers]
    out_xyz_ref, out_pts_ref = refs[2 * nlayers], refs[2 * nlayers + 1]
    gn_ref, gf_ref = refs[2 * nlayers + 2], refs[2 * nlayers + 3]
    xyz = xt_ref[0]
    Fm = f_ref[0]
    nx = _fps_body(xyz, npoint)
    out_xyz_ref[0] = nx
    nxT = jnp.transpose(nx)
    d2 = _sqdists(nxT, xyz)

    def slot(k, has, sel_f):
        gT = jnp.transpose(_mm_t(xyz, sel_f, _HI)) - nxT
        gn_ref[k] = jnp.where(has, gT, gn_ref[0])
        fk = _mm(sel_f, Fm, _HI)
        gf_ref[k] = jnp.where(has, fk, gf_ref[0])

    _ball_gather(d2, radius, nsample, slot)
    cf = Fm.shape[1]
    X = jnp.concatenate([gn_ref[...].reshape(nsample * npoint, 3),
                         gf_ref[...].reshape(nsample * npoint, cf)], axis=1)
    wbs = [(prs[2 * i][...], prs[2 * i + 1][...]) for i in range(nlayers)]
    h = _mlp(X, wbs, True)
    out_pts_ref[0] = jnp.max(h.reshape(nsample, npoint, h.shape[1]), axis=0)


def _fp_kernel(x1t_ref, x2t_ref, p2_ref, *refs, has_p1, nlayers):
    i = 1 if has_p1 else 0
    prs = refs[i:i + 2 * nlayers]
    out_ref = refs[i + 2 * nlayers]
    interp = _interp3(x1t_ref[0], x2t_ref[0], p2_ref[0])
    if has_p1:
        X = jnp.concatenate([refs[0][0], interp], axis=1)
    else:
        X = interp
    wbs = [(prs[2 * i_][...], prs[2 * i_ + 1][...]) for i_ in range(nlayers)]
    out_ref[0] = _mlp(X, wbs, True)


def _fp1_heads_kernel(x1t_ref, x2t_ref, p2_ref, *refs, n_fp, n_sem, n_off):
    nw = n_fp + n_sem + n_off
    prs = refs[:2 * nw]
    out_sem_ref, out_off_ref = refs[2 * nw], refs[2 * nw + 1]
    interp = _interp3(x1t_ref[0], x2t_ref[0], p2_ref[0])
    wbs = [(prs[2 * i][...], prs[2 * i + 1][...]) for i in range(nw)]
    feat = _mlp(interp, wbs[:n_fp], True)
    out_sem_ref[0] = jnp.transpose(_mlp(feat, wbs[n_fp:n_fp + n_sem], False))
    out_off_ref[0] = jnp.transpose(_mlp(feat, wbs[n_fp + n_sem:], False))


def _wspecs(layers, arity):
    ops, specs = [], []
    if arity == 1:
        zmap = lambda b: (0, 0)
    else:
        zmap = lambda b, j: (0, 0)
    for w, b in layers:
        ops += [w, b]
        specs += [pl.BlockSpec(w.shape, zmap), pl.BlockSpec(b.shape, zmap)]
    return ops, specs


def kernel(coords, feats, params):
    del feats
    B, N, _ = coords.shape
    xt = jnp.transpose(coords, (0, 2, 1))
    pr = {k: [(w, b.reshape(1, -1)) for (w, b) in v] for k, v in params.items()}

    (P1, R1, K1), cfg2, cfg3, cfg4 = _SA_CFG

    l1_xt = pl.pallas_call(
        functools.partial(_fps1_kernel, npoint=P1),
        grid=(B,),
        in_specs=[pl.BlockSpec((1, 3, N), lambda b: (b, 0, 0))],
        out_specs=pl.BlockSpec((1, 3, P1), lambda b: (b, 0, 0)),
        out_shape=jax.ShapeDtypeStruct((B, 3, P1), F32),
    )(xt)

    SB = _SA1_BLOCK
    l1_x = jnp.transpose(l1_xt, (0, 2, 1))
    ops, specs = _wspecs(pr['sa1'], 2)
    c1 = pr['sa1'][-1][0].shape[0]
    l1_pts = pl.pallas_call(
        functools.partial(_sa1_kernel, radius=R1, nsample=K1,
                          nlayers=len(pr['sa1'])),
        grid=(B, P1 // SB),
        in_specs=[pl.BlockSpec((1, 3, N), lambda b, j: (b, 0, 0)),
                  pl.BlockSpec((1, SB, 3), lambda b, j: (b, j, 0))] + specs,
        out_specs=pl.BlockSpec((1, SB, c1), lambda b, j: (b, j, 0)),
        out_shape=jax.ShapeDtypeStruct((B, P1, c1), F32),
        scratch_shapes=[pltpu.VMEM((K1, SB, 3), F32)],
    )(xt, l1_x, *ops)

    def sa_level(xt_in, pts_in, name, cfg):
        npoint, radius, nsample = cfg
        layers = pr[name]
        n_in, cf = xt_in.shape[2], pts_in.shape[2]
        cout = layers[-1][0].shape[0]
        ops, specs = _wspecs(layers, 1)
        return pl.pallas_call(
            functools.partial(_sa_full_kernel, npoint=npoint, radius=radius,
                              nsample=nsample, nlayers=len(layers)),
            grid=(B,),
            in_specs=[pl.BlockSpec((1, 3, n_in), lambda b: (b, 0, 0)),
                      pl.BlockSpec((1, n_in, cf), lambda b: (b, 0, 0))] + specs,
            out_specs=[pl.BlockSpec((1, 3, npoint), lambda b: (b, 0, 0)),
                       pl.BlockSpec((1, npoint, cout), lambda b: (b, 0, 0))],
            out_shape=[jax.ShapeDtypeStruct((B, 3, npoint), F32),
                       jax.ShapeDtypeStruct((B, npoint, cout), F32)],
            scratch_shapes=[pltpu.VMEM((nsample, npoint, 3), F32),
                            pltpu.VMEM((nsample, npoint, cf), F32)],
        )(xt_in, pts_in, *ops)

    l2_xt, l2_pts = sa_level(l1_xt, l1_pts, 'sa2', cfg2)
    l3_xt, l3_pts = sa_level(l2_xt, l2_pts, 'sa3', cfg3)
    l4_xt, l4_pts = sa_level(l3_xt, l3_pts, 'sa4', cfg4)

    def fp_level(x1t, x2t, p1, p2, name):
        layers = pr[name]
        n1, n2, c2 = x1t.shape[2], x2t.shape[2], p2.shape[2]
        cout = layers[-1][0].shape[0]
        ops = [x1t, x2t, p2]
        specs = [pl.BlockSpec((1, 3, n1), lambda b: (b, 0, 0)),
                 pl.BlockSpec((1, 3, n2), lambda b: (b, 0, 0)),
                 pl.BlockSpec((1, n2, c2), lambda b: (b, 0, 0))]
        if p1 is not None:
            ops.append(p1)
            specs.append(pl.BlockSpec((1, n1, p1.shape[2]), lambda b: (b, 0, 0)))
        wops, wsp = _wspecs(layers, 1)
        return pl.pallas_call(
            functools.partial(_fp_kernel, has_p1=p1 is not None,
                              nlayers=len(layers)),
            grid=(B,),
            in_specs=specs + wsp,
            out_specs=pl.BlockSpec((1, n1, cout), lambda b: (b, 0, 0)),
            out_shape=jax.ShapeDtypeStruct((B, n1, cout), F32),
        )(*(ops + wops))

    l3_fp = fp_level(l3_xt, l4_xt, l3_pts, l4_pts, 'fp4')
    l2_fp = fp_level(l2_xt, l3_xt, l2_pts, l3_fp, 'fp3')
    l1_fp = fp_level(l1_xt, l2_xt, l1_pts, l2_fp, 'fp2')

    SB1 = _FP1_BLOCK
    n_fp, n_sem, n_off = len(pr['fp1']), len(pr['sem']), len(pr['off'])
    c2 = l1_fp.shape[2]
    wops, wsp = _wspecs(pr['fp1'] + pr['sem'] + pr['off'], 2)
    sem, off = pl.pallas_call(
        functools.partial(_fp1_heads_kernel, n_fp=n_fp, n_sem=n_sem,
                          n_off=n_off),
        grid=(B, N // SB1),
        in_specs=[pl.BlockSpec((1, 3, SB1), lambda b, j: (b, 0, j)),
                  pl.BlockSpec((1, 3, P1), lambda b, j: (b, 0, 0)),
                  pl.BlockSpec((1, P1, c2), lambda b, j: (b, 0, 0))] + wsp,
        out_specs=[pl.BlockSpec((1, 2, SB1), lambda b, j: (b, 0, j)),
                   pl.BlockSpec((1, 3, SB1), lambda b, j: (b, 0, j))],
        out_shape=[jax.ShapeDtypeStruct((B, 2, N), F32),
                   jax.ShapeDtypeStruct((B, 3, N), F32)],
    )(xt, l1_xt, l1_fp, *wops)
    return sem, off
